# baseline (speedup 1.0000x reference)
import functools

import jax
import jax.numpy as jnp
from jax import lax
from jax.experimental import pallas as pl
from jax.experimental.pallas import tpu as pltpu
from jax.experimental.pallas import tpu_sc as plsc

NBUF = 4


@functools.partial(jax.jit, static_argnums=(2, 3, 4))
def _sc_gather(ids4d, table, bsz, seq, d):
    info = plsc.get_sparse_core_info()
    nw = info.num_cores * info.num_subcores
    sblk = seq // 8
    nbb = bsz // 128
    ndb = d // 8
    assert nbb == nw and seq % NBUF == 0
    mesh = plsc.VectorSubcoreMesh(core_axis_name="c", subcore_axis_name="s")

    @functools.partial(
        pl.kernel,
        mesh=mesh,
        out_type=jax.ShapeDtypeStruct((seq, ndb, nbb, 8, 128), jnp.float32),
        scratch_types=[
            pltpu.VMEM((sblk, 8, 128), jnp.int32),
            pltpu.VMEM((NBUF, 128, d), jnp.float32),
            pltpu.VMEM((NBUF, ndb, 8, 128), jnp.float32),
        ]
        + [pltpu.SemaphoreType.DMA] * (2 * NBUF),
        compiler_params=pltpu.CompilerParams(
            use_tc_tiling_on_sc=False, needs_layout_passes=False
        ),
    )
    def k(ids_hbm, table_hbm, out_hbm, idx_v, rows_v, rowsT_v, *sems):
        gsem = sems[:NBUF]
        osem = sems[NBUF:]
        bb = lax.axis_index("s") * info.num_cores + lax.axis_index("c")
        iota16 = jnp.arange(16, dtype=jnp.int32)

        def stage(sb, carry):
            pltpu.sync_copy(ids_hbm.at[sb, bb], idx_v.at[sb])
            return carry

        lax.fori_loop(0, sblk, stage, 0)

        def gather_fire(s, b):
            pltpu.async_copy(
                table_hbm.at[idx_v.at[s // 8, s % 8]], rows_v.at[b], gsem[b]
            )

        def gather_wait(b):
            pltpu.make_async_copy(
                table_hbm.at[idx_v.at[0, 0]], rows_v.at[b], gsem[b]
            ).wait()

        def stores_fire(s, b):
            for db in range(ndb):
                pltpu.async_copy(
                    rowsT_v.at[b].at[db], out_hbm.at[s, db, bb], osem[b]
                )

        def stores_wait(b):
            pltpu.make_async_copy(
                out_hbm.at[0, 0, 0], rowsT_v.at[b], osem[b]
            ).wait()

        def transpose(b):
            for dd in range(d):
                db = dd // 8
                r2 = dd % 8
                col = jnp.full((16,), dd, dtype=jnp.int32)
                for h in range(8):
                    vals = plsc.load_gather(
                        rows_v.at[b], [iota16 + (16 * h), col]
                    )
                    rowsT_v[b, db, r2, pl.ds(16 * h, 16)] = vals

        for b in range(NBUF):
            gather_fire(b, b)

        def body(g, carry):
            for b in range(NBUF):
                s = g * NBUF + b
                gather_wait(b)

                @pl.when(g > 0)
                def _():
                    stores_wait(b)

                transpose(b)
                stores_fire(s, b)

                @pl.when(s + NBUF < seq)
                def _():
                    gather_fire(s + NBUF, b)

            return carry

        lax.fori_loop(0, seq // NBUF, body, 0)
        for b in range(NBUF):
            stores_wait(b)

    return k(ids4d, table)


def kernel(input_ids, table):
    bsz, seq = input_ids.shape
    d = table.shape[1]
    ids4d = (
        input_ids.T.reshape(seq // 8, 8, bsz // 128, 128)
        .transpose(0, 2, 1, 3)
        .astype(jnp.int32)
    )
    out5d = _sc_gather(ids4d, table, bsz, seq, d)
    return out5d.transpose(2, 4, 0, 1, 3).reshape(bsz, seq, d)


# speedup vs baseline: 1.4632x; 1.4632x over previous
import functools

import jax
import jax.numpy as jnp
from jax import lax
from jax.experimental import pallas as pl
from jax.experimental.pallas import tpu as pltpu
from jax.experimental.pallas import tpu_sc as plsc

NBUF = 4


@functools.partial(jax.jit, static_argnums=(2, 3, 4))
def _sc_gather(ids4d, table, bsz, seq, d):
    info = plsc.get_sparse_core_info()
    nw = info.num_cores * info.num_subcores
    sblk = seq // 8
    nbb = bsz // 128
    ndb = d // 8
    assert nbb == nw and seq % NBUF == 0
    mesh = plsc.VectorSubcoreMesh(core_axis_name="c", subcore_axis_name="s")

    @functools.partial(
        pl.kernel,
        mesh=mesh,
        out_type=jax.ShapeDtypeStruct((seq, ndb, nbb, 8, 128), jnp.float32),
        scratch_types=[
            pltpu.VMEM((sblk, 8, 128), jnp.int32),
            pltpu.VMEM((NBUF, 128, d), jnp.float32),
            pltpu.VMEM((NBUF, ndb, 8, 128), jnp.float32),
        ]
        + [pltpu.SemaphoreType.DMA] * (2 * NBUF),
        compiler_params=pltpu.CompilerParams(
            use_tc_tiling_on_sc=False, needs_layout_passes=False
        ),
    )
    def k(ids_hbm, table_hbm, out_hbm, idx_v, rows_v, rowsT_v, *sems):
        gsem = sems[:NBUF]
        osem = sems[NBUF:]
        bb = lax.axis_index("s") * info.num_cores + lax.axis_index("c")
        iota16 = jnp.arange(16, dtype=jnp.int32)

        def stage(sb, carry):
            pltpu.sync_copy(ids_hbm.at[sb, bb], idx_v.at[sb])
            return carry

        lax.fori_loop(0, sblk, stage, 0)

        def gather_fire(s, b):
            pltpu.async_copy(
                table_hbm.at[idx_v.at[s // 8, s % 8]], rows_v.at[b], gsem[b]
            )

        def gather_wait(b):
            pltpu.make_async_copy(
                table_hbm.at[idx_v.at[0, 0]], rows_v.at[b], gsem[b]
            ).wait()

        def stores_fire(s, b):
            for db in range(ndb):
                pltpu.async_copy(
                    rowsT_v.at[b].at[db], out_hbm.at[s, db, bb], osem[b]
                )

        def stores_wait(b):
            pltpu.make_async_copy(
                out_hbm.at[0, 0, 0], rowsT_v.at[b], osem[b]
            ).wait()

        def transpose(b):
            @plsc.parallel_loop(0, d, 1, unroll=8)
            def _(dd):
                db = dd // 8
                r2 = dd % 8
                col = jnp.full((16,), dd, dtype=jnp.int32)
                for h in range(8):
                    vals = plsc.load_gather(
                        rows_v.at[b], [iota16 + (16 * h), col]
                    )
                    rowsT_v[b, db, r2, pl.ds(16 * h, 16)] = vals

        for b in range(NBUF):
            gather_fire(b, b)

        def body(g, carry):
            for b in range(NBUF):
                s = g * NBUF + b
                gather_wait(b)

                @pl.when(g > 0)
                def _():
                    stores_wait(b)

                transpose(b)
                stores_fire(s, b)

                @pl.when(s + NBUF < seq)
                def _():
                    gather_fire(s + NBUF, b)

            return carry

        lax.fori_loop(0, seq // NBUF, body, 0)
        for b in range(NBUF):
            stores_wait(b)

    return k(ids4d, table)


def kernel(input_ids, table):
    bsz, seq = input_ids.shape
    d = table.shape[1]
    ids4d = (
        input_ids.T.reshape(seq // 8, 8, bsz // 128, 128)
        .transpose(0, 2, 1, 3)
        .astype(jnp.int32)
    )
    out5d = _sc_gather(ids4d, table, bsz, seq, d)
    return out5d.transpose(2, 4, 0, 1, 3).reshape(bsz, seq, d)


# speedup vs baseline: 1.4822x; 1.0130x over previous
import functools

import jax
import jax.numpy as jnp
from jax import lax
from jax.experimental import pallas as pl
from jax.experimental.pallas import tpu as pltpu
from jax.experimental.pallas import tpu_sc as plsc

NBUF = 4


@functools.partial(jax.jit, static_argnums=(2, 3, 4))
def _sc_gather(ids4d, table, bsz, seq, d):
    info = plsc.get_sparse_core_info()
    nw = info.num_cores * info.num_subcores
    sblk = seq // 8
    nbb = bsz // 128
    ndb = d // 8
    assert nbb == nw and seq % NBUF == 0
    mesh = plsc.VectorSubcoreMesh(core_axis_name="c", subcore_axis_name="s")

    @functools.partial(
        pl.kernel,
        mesh=mesh,
        out_type=jax.ShapeDtypeStruct((seq, ndb, nbb, 8, 128), jnp.float32),
        scratch_types=[
            pltpu.VMEM((sblk, 8, 128), jnp.int32),
            pltpu.VMEM((NBUF, 128, d), jnp.float32),
            pltpu.VMEM((NBUF, ndb, 8, 128), jnp.float32),
        ]
        + [pltpu.SemaphoreType.DMA] * (2 * NBUF),
        compiler_params=pltpu.CompilerParams(
            use_tc_tiling_on_sc=False, needs_layout_passes=False
        ),
    )
    def k(ids_hbm, table_hbm, out_hbm, idx_v, rows_v, rowsT_v, *sems):
        gsem = sems[:NBUF]
        osem = sems[NBUF:]
        bb = lax.axis_index("s") * info.num_cores + lax.axis_index("c")
        iota16 = jnp.arange(16, dtype=jnp.int32)

        def stage(sb, carry):
            pltpu.sync_copy(ids_hbm.at[sb, bb], idx_v.at[sb])
            return carry

        lax.fori_loop(0, sblk, stage, 0)

        def gather_fire(s, b):
            pltpu.async_copy(
                table_hbm.at[idx_v.at[s // 8, s % 8]], rows_v.at[b], gsem[b]
            )

        def gather_wait(b):
            pltpu.make_async_copy(
                table_hbm.at[idx_v.at[0, 0]], rows_v.at[b], gsem[b]
            ).wait()

        def stores_fire(s, b):
            for db in range(ndb):
                pltpu.async_copy(
                    rowsT_v.at[b].at[db], out_hbm.at[s, db, bb], osem[b]
                )

        def stores_wait(b):
            pltpu.make_async_copy(
                out_hbm.at[0, 0, 0], rowsT_v.at[b], osem[b]
            ).wait()

        def transpose(b):
            @plsc.parallel_loop(0, d, 1, unroll=8)
            def _(dd):
                db = dd // 8
                r2 = dd % 8
                col = jnp.full((16,), dd, dtype=jnp.int32)
                for h in range(8):
                    vals = plsc.load_gather(
                        rows_v.at[b], [iota16 + (16 * h), col]
                    )
                    rowsT_v[b, db, r2, pl.ds(16 * h, 16)] = vals

        for b in range(NBUF):
            gather_fire(b, b)

        def body(g, carry):
            for b in range(NBUF):
                s = g * NBUF + b
                gather_wait(b)

                @pl.when(g > 0)
                def _():
                    stores_wait(b)

                transpose(b)
                stores_fire(s, b)

                @pl.when(s + NBUF < seq)
                def _():
                    gather_fire(s + NBUF, b)

            return carry

        lax.fori_loop(0, seq // NBUF, body, 0)
        for b in range(NBUF):
            stores_wait(b)

    return k(ids4d, table)


def kernel(input_ids, table):
    bsz, seq = input_ids.shape
    d = table.shape[1]
    scale = 128 // d
    tpad = jnp.pad(table, ((0, 0), (0, 128 - d))).reshape(-1, d)
    ids4d = (
        input_ids.T.reshape(seq // 8, 8, bsz // 128, 128)
        .transpose(0, 2, 1, 3)
        .astype(jnp.int32)
    ) * scale
    out5d = _sc_gather(ids4d, tpad, bsz, seq, d)
    return out5d.transpose(2, 4, 0, 1, 3).reshape(bsz, seq, d)


# speedup vs baseline: 1.6446x; 1.1096x over previous
import functools

import jax
import jax.numpy as jnp
from jax import lax
from jax.experimental import pallas as pl
from jax.experimental.pallas import tpu as pltpu
from jax.experimental.pallas import tpu_sc as plsc

NBUF = 4


@functools.partial(jax.jit, static_argnums=(2, 3, 4))
def _sc_gather(ids4d, table, bsz, seq, d):
    info = plsc.get_sparse_core_info()
    nw = info.num_cores * info.num_subcores
    sblk = seq // 8
    nbb = bsz // 128
    ndb = d // 8
    assert nbb == nw and seq % NBUF == 0
    mesh = plsc.VectorSubcoreMesh(core_axis_name="c", subcore_axis_name="s")

    @functools.partial(
        pl.kernel,
        mesh=mesh,
        out_type=jax.ShapeDtypeStruct((seq, ndb, nbb, 8, 128), jnp.float32),
        scratch_types=[
            pltpu.VMEM((sblk, 8, 128), jnp.int32),
            pltpu.VMEM((NBUF, 128, d), jnp.float32),
            pltpu.VMEM((NBUF, ndb, 8, 128), jnp.float32),
        ]
        + [pltpu.SemaphoreType.DMA] * (2 * NBUF),
        compiler_params=pltpu.CompilerParams(
            use_tc_tiling_on_sc=False, needs_layout_passes=False
        ),
    )
    def k(ids_hbm, table_hbm, out_hbm, idx_v, rows_v, rowsT_v, *sems):
        gsem = sems[:NBUF]
        osem = sems[NBUF:]
        bb = lax.axis_index("s") * info.num_cores + lax.axis_index("c")
        iota16 = jnp.arange(16, dtype=jnp.int32)

        def stage(sb, carry):
            pltpu.sync_copy(ids_hbm.at[sb, bb], idx_v.at[sb])
            return carry

        lax.fori_loop(0, sblk, stage, 0)

        def gather_fire(s, b):
            pltpu.async_copy(
                table_hbm.at[idx_v.at[s // 8, s % 8]], rows_v.at[b], gsem[b]
            )

        def gather_wait(b):
            pltpu.make_async_copy(
                table_hbm.at[idx_v.at[0, 0]], rows_v.at[b], gsem[b]
            ).wait()

        def stores_fire(s, b):
            for db in range(ndb):
                pltpu.async_copy(
                    rowsT_v.at[b].at[db], out_hbm.at[s, db, bb], osem[b]
                )

        def stores_wait(b):
            pltpu.make_async_copy(
                out_hbm.at[0, 0, 0], rowsT_v.at[b], osem[b]
            ).wait()

        def transpose(b):
            @plsc.parallel_loop(0, d, 1, unroll=8)
            def _(dd):
                db = dd // 8
                r2 = dd % 8
                col = jnp.full((16,), dd, dtype=jnp.int32)
                for h in range(8):
                    vals = plsc.load_gather(
                        rows_v.at[b], [iota16 + (16 * h), col]
                    )
                    rowsT_v[b, db, r2, pl.ds(16 * h, 16)] = vals

        for b in range(NBUF):
            gather_fire(b, b)

        def body(g, carry):
            for b in range(NBUF):
                s = g * NBUF + b
                gather_wait(b)

                @pl.when(g > 0)
                def _():
                    stores_wait(b)

                transpose(b)
                stores_fire(s, b)

                @pl.when(s + NBUF < seq)
                def _():
                    gather_fire(s + NBUF, b)

            return carry

        lax.fori_loop(0, seq // NBUF, body, 0)
        for b in range(NBUF):
            stores_wait(b)

    return k(ids4d, table)


def _tc_linearize(tT, n, d):
    rpo = 128 // d
    blk = 4096
    grid = (n + blk - 1) // blk

    def body(in_ref, out_ref):
        xt = in_ref[...].T
        y = xt.reshape(blk // rpo, rpo, d)
        out_ref[...] = jnp.concatenate([y[:, a, :] for a in range(rpo)], axis=1)

    return pl.pallas_call(
        body,
        grid=(grid,),
        in_specs=[pl.BlockSpec((d, blk), lambda j: (0, j))],
        out_specs=pl.BlockSpec((blk // rpo, 128), lambda j: (j, 0)),
        out_shape=jax.ShapeDtypeStruct((n * d // 128, 128), jnp.float32),
    )(tT)


def kernel(input_ids, table):
    bsz, seq = input_ids.shape
    n, d = table.shape
    tlin = _tc_linearize(table.T, n, d).reshape(n, d)
    ids4d = (
        input_ids.T.reshape(seq // 8, 8, bsz // 128, 128)
        .transpose(0, 2, 1, 3)
        .astype(jnp.int32)
    )
    out5d = _sc_gather(ids4d, tlin, bsz, seq, d)
    return out5d.transpose(2, 4, 0, 1, 3).reshape(bsz, seq, d)


# speedup vs baseline: 1.6679x; 1.0142x over previous
import functools

import jax
import jax.numpy as jnp
from jax import lax
from jax.experimental import pallas as pl
from jax.experimental.pallas import tpu as pltpu
from jax.experimental.pallas import tpu_sc as plsc

NBUF = 8


@functools.partial(jax.jit, static_argnums=(2, 3, 4))
def _sc_gather(ids4d, table, bsz, seq, d):
    info = plsc.get_sparse_core_info()
    nw = info.num_cores * info.num_subcores
    sblk = seq // 8
    nbb = bsz // 128
    ndb = d // 8
    assert nbb == nw and seq % NBUF == 0
    mesh = plsc.VectorSubcoreMesh(core_axis_name="c", subcore_axis_name="s")

    @functools.partial(
        pl.kernel,
        mesh=mesh,
        out_type=jax.ShapeDtypeStruct((seq, ndb, nbb, 8, 128), jnp.float32),
        scratch_types=[
            pltpu.VMEM((sblk, 8, 128), jnp.int32),
            pltpu.VMEM((NBUF, 128, d), jnp.float32),
            pltpu.VMEM((NBUF, ndb, 8, 128), jnp.float32),
        ]
        + [pltpu.SemaphoreType.DMA] * (2 * NBUF),
        compiler_params=pltpu.CompilerParams(
            use_tc_tiling_on_sc=False, needs_layout_passes=False
        ),
    )
    def k(ids_hbm, table_hbm, out_hbm, idx_v, rows_v, rowsT_v, *sems):
        gsem = sems[:NBUF]
        osem = sems[NBUF:]
        bb = lax.axis_index("s") * info.num_cores + lax.axis_index("c")
        iota16 = jnp.arange(16, dtype=jnp.int32)

        def stage(sb, carry):
            pltpu.sync_copy(ids_hbm.at[sb, bb], idx_v.at[sb])
            return carry

        lax.fori_loop(0, sblk, stage, 0)

        def gather_fire(s, b):
            pltpu.async_copy(
                table_hbm.at[idx_v.at[s // 8, s % 8]], rows_v.at[b], gsem[b]
            )

        def gather_wait(b):
            pltpu.make_async_copy(
                table_hbm.at[idx_v.at[0, 0]], rows_v.at[b], gsem[b]
            ).wait()

        def stores_fire(s, b):
            for db in range(ndb):
                pltpu.async_copy(
                    rowsT_v.at[b].at[db], out_hbm.at[s, db, bb], osem[b]
                )

        def stores_wait(b):
            pltpu.make_async_copy(
                out_hbm.at[0, 0, 0], rowsT_v.at[b], osem[b]
            ).wait()

        def transpose(b):
            @plsc.parallel_loop(0, d, 1, unroll=8)
            def _(dd):
                db = dd // 8
                r2 = dd % 8
                col = jnp.full((16,), dd, dtype=jnp.int32)
                for h in range(8):
                    vals = plsc.load_gather(
                        rows_v.at[b], [iota16 + (16 * h), col]
                    )
                    rowsT_v[b, db, r2, pl.ds(16 * h, 16)] = vals

        for b in range(NBUF):
            gather_fire(b, b)

        def body(g, carry):
            for b in range(NBUF):
                s = g * NBUF + b
                gather_wait(b)

                @pl.when(g > 0)
                def _():
                    stores_wait(b)

                transpose(b)
                stores_fire(s, b)

                @pl.when(s + NBUF < seq)
                def _():
                    gather_fire(s + NBUF, b)

            return carry

        lax.fori_loop(0, seq // NBUF, body, 0)
        for b in range(NBUF):
            stores_wait(b)

    return k(ids4d, table)


def _tc_linearize(tT, n, d):
    rpo = 128 // d
    blk = 4096
    grid = (n + blk - 1) // blk

    def body(in_ref, out_ref):
        xt = in_ref[...].T
        y = xt.reshape(blk // rpo, rpo, d)
        for a in range(rpo):
            out_ref[:, a * d : (a + 1) * d] = y[:, a, :]

    return pl.pallas_call(
        body,
        grid=(grid,),
        in_specs=[pl.BlockSpec((d, blk), lambda j: (0, j))],
        out_specs=pl.BlockSpec((blk // rpo, 128), lambda j: (j, 0)),
        out_shape=jax.ShapeDtypeStruct((n * d // 128, 128), jnp.float32),
        compiler_params=pltpu.CompilerParams(
            dimension_semantics=("parallel",)
        ),
    )(tT)


def kernel(input_ids, table):
    bsz, seq = input_ids.shape
    n, d = table.shape
    tlin = _tc_linearize(table.T, n, d).reshape(n, d)
    ids4d = (
        input_ids.T.reshape(seq // 8, 8, bsz // 128, 128)
        .transpose(0, 2, 1, 3)
        .astype(jnp.int32)
    )
    out5d = _sc_gather(ids4d, tlin, bsz, seq, d)
    return out5d.transpose(2, 4, 0, 1, 3).reshape(bsz, seq, d)


# speedup vs baseline: 1.6824x; 1.0086x over previous
import functools

import jax
import jax.numpy as jnp
from jax import lax
from jax.experimental import pallas as pl
from jax.experimental.pallas import tpu as pltpu
from jax.experimental.pallas import tpu_sc as plsc

NBUF = 4


@functools.partial(jax.jit, static_argnums=(2, 3, 4))
def _sc_gather(ids4d, table, bsz, seq, d):
    info = plsc.get_sparse_core_info()
    nw = info.num_cores * info.num_subcores
    sblk = seq // 8
    nbb = bsz // 128
    ndb = d // 8
    assert nbb == nw and seq % NBUF == 0
    mesh = plsc.VectorSubcoreMesh(core_axis_name="c", subcore_axis_name="s")

    @functools.partial(
        pl.kernel,
        mesh=mesh,
        out_type=jax.ShapeDtypeStruct((seq, ndb, nbb, 8, 128), jnp.float32),
        scratch_types=[
            pltpu.VMEM((sblk, 8, 128), jnp.int32),
            pltpu.VMEM((NBUF, 128, d), jnp.float32),
            pltpu.VMEM((NBUF, ndb, 8, 128), jnp.float32),
        ]
        + [pltpu.SemaphoreType.DMA] * (2 * NBUF),
        compiler_params=pltpu.CompilerParams(
            use_tc_tiling_on_sc=False, needs_layout_passes=False
        ),
    )
    def k(ids_hbm, table_hbm, out_hbm, idx_v, rows_v, rowsT_v, *sems):
        gsem = sems[:NBUF]
        osem = sems[NBUF:]
        bb = lax.axis_index("s") * info.num_cores + lax.axis_index("c")
        iota16 = jnp.arange(16, dtype=jnp.int32)

        def stage(sb, carry):
            pltpu.sync_copy(ids_hbm.at[sb, bb], idx_v.at[sb])
            return carry

        lax.fori_loop(0, sblk, stage, 0)

        def gather_fire(s, b):
            pltpu.async_copy(
                table_hbm.at[idx_v.at[s // 8, s % 8]], rows_v.at[b], gsem[b]
            )

        def gather_wait(b):
            pltpu.make_async_copy(
                table_hbm.at[idx_v.at[0, 0]], rows_v.at[b], gsem[b]
            ).wait()

        def stores_fire(s, b):
            for db in range(ndb):
                pltpu.async_copy(
                    rowsT_v.at[b].at[db], out_hbm.at[s, db, bb], osem[b]
                )

        def stores_wait(b):
            pltpu.make_async_copy(
                out_hbm.at[0, 0, 0], rowsT_v.at[b], osem[b]
            ).wait()

        def transpose(b):
            @plsc.parallel_loop(0, d, 1, unroll=8)
            def _(dd):
                db = dd // 8
                r2 = dd % 8
                col = jnp.full((16,), dd, dtype=jnp.int32)
                for h in range(8):
                    vals = plsc.load_gather(
                        rows_v.at[b], [iota16 + (16 * h), col]
                    )
                    rowsT_v[b, db, r2, pl.ds(16 * h, 16)] = vals

        for b in range(NBUF):
            gather_fire(b, b)

        def body(g, carry):
            for b in range(NBUF):
                s = g * NBUF + b
                gather_wait(b)

                @pl.when(g > 0)
                def _():
                    stores_wait(b)

                transpose(b)
                stores_fire(s, b)

                @pl.when(s + NBUF < seq)
                def _():
                    gather_fire(s + NBUF, b)

            return carry

        lax.fori_loop(0, seq // NBUF, body, 0)
        for b in range(NBUF):
            stores_wait(b)

    return k(ids4d, table)


def _tc_linearize(tT, n, d):
    rpo = 128 // d
    blk = 4096
    grid = (n + blk - 1) // blk

    def body(in_ref, out_ref):
        xt = in_ref[...].T
        y = xt.reshape(blk // rpo, rpo, d)
        for a in range(rpo):
            out_ref[:, a * d : (a + 1) * d] = y[:, a, :]

    return pl.pallas_call(
        body,
        grid=(grid,),
        in_specs=[pl.BlockSpec((d, blk), lambda j: (0, j))],
        out_specs=pl.BlockSpec((blk // rpo, 128), lambda j: (j, 0)),
        out_shape=jax.ShapeDtypeStruct((n * d // 128, 128), jnp.float32),
        compiler_params=pltpu.CompilerParams(
            dimension_semantics=("parallel",)
        ),
    )(tT)


def kernel(input_ids, table):
    bsz, seq = input_ids.shape
    n, d = table.shape
    tlin = _tc_linearize(table.T, n, d).reshape(n, d)
    ids4d = (
        input_ids.T.reshape(seq // 8, 8, bsz // 128, 128)
        .transpose(0, 2, 1, 3)
        .astype(jnp.int32)
    )
    out5d = _sc_gather(ids4d, tlin, bsz, seq, d)
    return out5d.transpose(2, 4, 0, 1, 3).reshape(bsz, seq, d)


# speedup vs baseline: 1.7589x; 1.0455x over previous
import functools

import jax
import jax.numpy as jnp
from jax import lax
from jax.experimental import pallas as pl
from jax.experimental.pallas import tpu as pltpu
from jax.experimental.pallas import tpu_sc as plsc

NBUF = 4


@functools.partial(jax.jit, static_argnums=(2, 3, 4))
def _sc_gather(ids4d, table, bsz, seq, d):
    info = plsc.get_sparse_core_info()
    nw = info.num_cores * info.num_subcores
    sblk = seq // 8
    nbb = bsz // 128
    ndb = d // 8
    assert nbb == nw and seq % NBUF == 0
    mesh = plsc.VectorSubcoreMesh(core_axis_name="c", subcore_axis_name="s")

    @functools.partial(
        pl.kernel,
        mesh=mesh,
        out_type=jax.ShapeDtypeStruct((seq, ndb, nbb, 8, 128), jnp.float32),
        scratch_types=[
            pltpu.VMEM((sblk, 8, 128), jnp.int32),
            pltpu.VMEM((NBUF, 128, d), jnp.float32),
            pltpu.VMEM((NBUF, ndb, 8, 128), jnp.float32),
        ]
        + [pltpu.SemaphoreType.DMA] * (2 * NBUF),
        compiler_params=pltpu.CompilerParams(
            use_tc_tiling_on_sc=False, needs_layout_passes=False
        ),
    )
    def k(ids_hbm, table_hbm, out_hbm, idx_v, rows_v, rowsT_v, *sems):
        gsem = sems[:NBUF]
        osem = sems[NBUF:]
        bb = lax.axis_index("s") * info.num_cores + lax.axis_index("c")
        iota16 = jnp.arange(16, dtype=jnp.int32)

        def stage(sb, carry):
            pltpu.sync_copy(ids_hbm.at[sb, bb], idx_v.at[sb])
            return carry

        lax.fori_loop(0, sblk, stage, 0)

        def gather_fire(s, b):
            pltpu.async_copy(
                table_hbm.at[idx_v.at[s // 8, s % 8]], rows_v.at[b], gsem[b]
            )

        def gather_wait(b):
            pltpu.make_async_copy(
                table_hbm.at[idx_v.at[0, 0]], rows_v.at[b], gsem[b]
            ).wait()

        def stores_fire(s, b):
            for db in range(ndb):
                pltpu.async_copy(
                    rowsT_v.at[b].at[db], out_hbm.at[s, db, bb], osem[b]
                )

        def stores_wait(b):
            pltpu.make_async_copy(
                out_hbm.at[0, 0, 0], rowsT_v.at[b], osem[b]
            ).wait()

        def transpose(b):
            @plsc.parallel_loop(0, d, 1, unroll=8)
            def _(dd):
                db = dd // 8
                r2 = dd % 8
                col = jnp.full((16,), dd, dtype=jnp.int32)
                for h in range(8):
                    vals = plsc.load_gather(
                        rows_v.at[b], [iota16 + (16 * h), col]
                    )
                    rowsT_v[b, db, r2, pl.ds(16 * h, 16)] = vals

        for b in range(NBUF):
            gather_fire(b, b)

        def body(g, carry):
            for b in range(NBUF):
                s = g * NBUF + b
                gather_wait(b)

                @pl.when(g > 0)
                def _():
                    stores_wait(b)

                transpose(b)
                stores_fire(s, b)

                @pl.when(s + NBUF < seq)
                def _():
                    gather_fire(s + NBUF, b)

            return carry

        lax.fori_loop(0, seq // NBUF, body, 0)
        for b in range(NBUF):
            stores_wait(b)

    return k(ids4d, table)


def _tc_linearize(tT, n, d):
    rpo = 128 // d
    blk = 16384
    grid = (n + blk - 1) // blk

    def body(in_ref, out_ref):
        xt = in_ref[...].T
        y = xt.reshape(blk // rpo, rpo, d)
        for a in range(rpo):
            out_ref[:, a * d : (a + 1) * d] = y[:, a, :]

    return pl.pallas_call(
        body,
        grid=(grid,),
        in_specs=[pl.BlockSpec((d, blk), lambda j: (0, j))],
        out_specs=pl.BlockSpec((blk // rpo, 128), lambda j: (j, 0)),
        out_shape=jax.ShapeDtypeStruct((n * d // 128, 128), jnp.float32),
        compiler_params=pltpu.CompilerParams(
            dimension_semantics=("parallel",)
        ),
    )(tT)


def kernel(input_ids, table):
    bsz, seq = input_ids.shape
    n, d = table.shape
    tlin = _tc_linearize(table.T, n, d).reshape(n, d)
    ids4d = (
        input_ids.T.reshape(seq // 8, 8, bsz // 128, 128)
        .transpose(0, 2, 1, 3)
        .astype(jnp.int32)
    )
    out5d = _sc_gather(ids4d, tlin, bsz, seq, d)
    return out5d.transpose(2, 4, 0, 1, 3).reshape(bsz, seq, d)
